# baseline (speedup 1.0000x reference)
import functools

import jax
import jax.numpy as jnp
from jax import lax
from jax.experimental import pallas as pl
from jax.experimental.pallas import tpu as pltpu
from jax.experimental.pallas import tpu_sc as plsc

NC = 2
NS = 16
L = 16
D = 128


@functools.lru_cache(maxsize=None)
def _make_sc_kernel(B: int):
    assert B % (8 * NC * NS) == 0 and D % L == 0
    b_per_w = B // (NC * NS)
    C = 64
    n_chunks = b_per_w // C
    CU = 8

    mesh = plsc.VectorSubcoreMesh(
        core_axis_name="c", subcore_axis_name="s",
        num_cores=NC, num_subcores=NS)

    scratch = []
    for _ in range(2):
        scratch += [
            pltpu.VMEM((C,), jnp.int32),
            pltpu.VMEM((C,), jnp.int32),
            pltpu.VMEM((C, D), jnp.float32),
            pltpu.VMEM((C, D), jnp.float32),
            pltpu.VMEM((C, D), jnp.float32),
            pltpu.VMEM((C, D), jnp.float32),
        ]
    scratch += [
        pltpu.VMEM((L,), jnp.float32),
        pltpu.VMEM((L,), jnp.int32),
        pltpu.VMEM((L,), jnp.float32),
        pltpu.SemaphoreType.DMA,
        pltpu.SemaphoreType.DMA,
        pltpu.SemaphoreType.DMA,
        pltpu.SemaphoreType.DMA,
    ]

    @functools.partial(
        pl.kernel,
        out_type=jax.ShapeDtypeStruct((NC * NS, L), jnp.float32),
        mesh=mesh,
        scratch_types=scratch,
        compiler_params=pltpu.CompilerParams(needs_layout_passes=False),
    )
    def sc_kernel(xs_hbm, xd_hbm, hni_hbm, marg_hbm, bs_hbm, out_hbm,
                  i1a, i2a, xsa, xda, g1a, g2a,
                  i1b, i2b, xsb, xdb, g1b, g2b,
                  marg_v, bs_v, acc_v, isem0, isem1, sem0, sem1):
        idx1_v = (i1a, i1b)
        idx2_v = (i2a, i2b)
        xs_v = (xsa, xsb)
        xd_v = (xda, xdb)
        g1_v = (g1a, g1b)
        g2_v = (g2a, g2b)
        isem = (isem0, isem1)
        sem = (sem0, sem1)

        wid = lax.axis_index("s") * NC + lax.axis_index("c")
        base = wid * b_per_w
        pltpu.sync_copy(marg_hbm, marg_v)
        pltpu.sync_copy(bs_hbm, bs_v)
        margin = marg_v[...]
        bs = bs_v[...]

        def stage_idx(ci, b):
            row0 = base + ci * C
            return (
                pltpu.async_copy(hni_hbm.at[pl.ds(row0, C)], idx1_v[b], isem[b]),
                pltpu.async_copy(hni_hbm.at[pl.ds(B + row0, C)], idx2_v[b], isem[b]),
            )

        def stage_bulk(ci, b):
            row0 = base + ci * C
            for j in range(C // L):
                sl = pl.ds(j * L, L)
                idx1_v[b][sl] = idx1_v[b][sl] - bs
            return (
                pltpu.async_copy(xs_hbm.at[pl.ds(row0, C)], xs_v[b], sem[b]),
                pltpu.async_copy(xd_hbm.at[pl.ds(row0, C)], xd_v[b], sem[b]),
                pltpu.async_copy(xd_hbm.at[idx1_v[b]], g1_v[b], sem[b]),
                pltpu.async_copy(xs_hbm.at[idx2_v[b]], g2_v[b], sem[b]),
            )

        def compute_chunk(b, acc):
            xs_r, xd_r, g1_r, g2_r = xs_v[b], xd_v[b], g1_v[b], g2_v[b]

            def group_body(g, a):
                rows = g * L + lax.iota(jnp.int32, L)

                def col_body(c8, carry):
                    p, n1, n2 = carry
                    for j in range(CU):
                        cols = jnp.full((L,), c8 * CU + j, jnp.int32)
                        s = plsc.load_gather(xs_r, [rows, cols])
                        t = plsc.load_gather(xd_r, [rows, cols])
                        a1 = plsc.load_gather(g1_r, [rows, cols])
                        a2 = plsc.load_gather(g2_r, [rows, cols])
                        dp = s - t
                        p = p + dp * dp
                        d1 = s - a1
                        n1 = n1 + d1 * d1
                        d2 = t - a2
                        n2 = n2 + d2 * d2
                    return p, n1, n2

                z = jnp.zeros((L,), jnp.float32)
                p, n1, n2 = lax.fori_loop(0, D // CU, col_body, (z, z, z))
                l1 = jnp.maximum(p - n1 + margin, 0.0)
                l2 = jnp.maximum(p - n2 + margin, 0.0)
                return a + l1 + l2

            return lax.fori_loop(0, C // L, group_body, acc)

        ic = stage_idx(0, 0)
        ic[0].wait()
        ic[1].wait()
        bulk = stage_bulk(0, 0)
        icn = stage_idx(1, 1) if n_chunks > 1 else None

        acc = jnp.zeros((L,), jnp.float32)
        for i in range(n_chunks):
            p = i % 2
            bulk_next = None
            if i + 1 < n_chunks:
                icn[0].wait()
                icn[1].wait()
                bulk_next = stage_bulk(i + 1, 1 - p)
            for cp in bulk:
                cp.wait()
            if i + 2 < n_chunks:
                icn = stage_idx(i + 2, p)
            acc = compute_chunk(p, acc)
            bulk = bulk_next

        acc_v[...] = acc
        pltpu.sync_copy(acc_v, out_hbm.at[wid])

    return sc_kernel


def kernel(x_shape, x_desc, batch_size, margin, hard_neg_ind):
    B = x_shape.shape[0]
    hni = hard_neg_ind.astype(jnp.int32)
    marg = jnp.broadcast_to(jnp.asarray(margin, jnp.float32), (L,))
    bs = jnp.broadcast_to(jnp.asarray(batch_size, jnp.int32), (L,))
    partials = _make_sc_kernel(B)(x_shape, x_desc, hni, marg, bs)
    return jnp.sum(partials)


# speedup vs baseline: 3.4325x; 3.4325x over previous
import functools

import jax
import jax.numpy as jnp
from jax import lax
from jax.experimental import pallas as pl
from jax.experimental.pallas import tpu as pltpu
from jax.experimental.pallas import tpu_sc as plsc

NC = 2
NS = 16
L = 16
D = 128


@functools.lru_cache(maxsize=None)
def _make_sc_kernel(B: int):
    assert B % (8 * NC * NS) == 0 and D % L == 0
    b_per_w = B // (NC * NS)
    C = 64
    n_chunks = b_per_w // C
    ROWS_U = 2

    mesh = plsc.VectorSubcoreMesh(
        core_axis_name="c", subcore_axis_name="s",
        num_cores=NC, num_subcores=NS)

    scratch = []
    for _ in range(2):
        scratch += [
            pltpu.VMEM((C,), jnp.int32),
            pltpu.VMEM((C,), jnp.int32),
            pltpu.VMEM((C, D), jnp.float32),
            pltpu.VMEM((C, D), jnp.float32),
            pltpu.VMEM((C, D), jnp.float32),
            pltpu.VMEM((C, D), jnp.float32),
        ]
    scratch += [
        pltpu.VMEM((L,), jnp.float32),
        pltpu.VMEM((L,), jnp.int32),
        pltpu.VMEM((L,), jnp.float32),
        pltpu.SemaphoreType.DMA,
        pltpu.SemaphoreType.DMA,
        pltpu.SemaphoreType.DMA,
        pltpu.SemaphoreType.DMA,
    ]

    @functools.partial(
        pl.kernel,
        out_type=jax.ShapeDtypeStruct((NC * NS, L), jnp.float32),
        mesh=mesh,
        scratch_types=scratch,
        compiler_params=pltpu.CompilerParams(needs_layout_passes=False),
    )
    def sc_kernel(xs_hbm, xd_hbm, hni_hbm, marg_hbm, bs_hbm, out_hbm,
                  i1a, i2a, xsa, xda, g1a, g2a,
                  i1b, i2b, xsb, xdb, g1b, g2b,
                  marg_v, bs_v, acc_v, isem0, isem1, sem0, sem1):
        idx1_v = (i1a, i1b)
        idx2_v = (i2a, i2b)
        xs_v = (xsa, xsb)
        xd_v = (xda, xdb)
        g1_v = (g1a, g1b)
        g2_v = (g2a, g2b)
        isem = (isem0, isem1)
        sem = (sem0, sem1)

        wid = lax.axis_index("s") * NC + lax.axis_index("c")
        base = wid * b_per_w
        pltpu.sync_copy(marg_hbm, marg_v)
        pltpu.sync_copy(bs_hbm, bs_v)
        margin = marg_v[...]
        bs = bs_v[...]
        last_lane = lax.iota(jnp.int32, L) == (L - 1)

        def stage_idx(ci, b):
            row0 = base + ci * C
            return (
                pltpu.async_copy(hni_hbm.at[pl.ds(row0, C)], idx1_v[b], isem[b]),
                pltpu.async_copy(hni_hbm.at[pl.ds(B + row0, C)], idx2_v[b], isem[b]),
            )

        def stage_bulk(ci, b):
            row0 = base + ci * C
            for j in range(C // L):
                sl = pl.ds(j * L, L)
                idx1_v[b][sl] = idx1_v[b][sl] - bs
            return (
                pltpu.async_copy(xs_hbm.at[pl.ds(row0, C)], xs_v[b], sem[b]),
                pltpu.async_copy(xd_hbm.at[pl.ds(row0, C)], xd_v[b], sem[b]),
                pltpu.async_copy(xd_hbm.at[idx1_v[b]], g1_v[b], sem[b]),
                pltpu.async_copy(xs_hbm.at[idx2_v[b]], g2_v[b], sem[b]),
            )

        def compute_chunk(b, acc):
            xs_r, xd_r, g1_r, g2_r = xs_v[b], xd_v[b], g1_v[b], g2_v[b]

            def pair_body(i, a):
                for u in range(ROWS_U):
                    r = i * ROWS_U + u
                    z = jnp.zeros((L,), jnp.float32)
                    p, n1, n2 = z, z, z
                    for k in range(D // L):
                        sl = pl.ds(k * L, L)
                        s = xs_r[r, sl]
                        t = xd_r[r, sl]
                        a1 = g1_r[r, sl]
                        a2 = g2_r[r, sl]
                        dp = s - t
                        p = p + dp * dp
                        d1 = s - a1
                        n1 = n1 + d1 * d1
                        d2 = t - a2
                        n2 = n2 + d2 * d2
                    c1 = plsc.cumsum(p - n1)
                    c2 = plsc.cumsum(p - n2)
                    l1 = jnp.maximum(c1 + margin, 0.0)
                    l2 = jnp.maximum(c2 + margin, 0.0)
                    a = a + jnp.where(last_lane, l1 + l2, 0.0)
                return a

            return lax.fori_loop(0, C // ROWS_U, pair_body, acc)

        ic = stage_idx(0, 0)
        ic[0].wait()
        ic[1].wait()
        bulk = stage_bulk(0, 0)
        icn = stage_idx(1, 1) if n_chunks > 1 else None

        acc = jnp.zeros((L,), jnp.float32)
        for i in range(n_chunks):
            p = i % 2
            bulk_next = None
            if i + 1 < n_chunks:
                icn[0].wait()
                icn[1].wait()
                bulk_next = stage_bulk(i + 1, 1 - p)
            for cp in bulk:
                cp.wait()
            if i + 2 < n_chunks:
                icn = stage_idx(i + 2, p)
            acc = compute_chunk(p, acc)
            bulk = bulk_next

        acc_v[...] = acc
        pltpu.sync_copy(acc_v, out_hbm.at[wid])

    return sc_kernel


def kernel(x_shape, x_desc, batch_size, margin, hard_neg_ind):
    B = x_shape.shape[0]
    hni = hard_neg_ind.astype(jnp.int32)
    marg = jnp.broadcast_to(jnp.asarray(margin, jnp.float32), (L,))
    bs = jnp.broadcast_to(jnp.asarray(batch_size, jnp.int32), (L,))
    partials = _make_sc_kernel(B)(x_shape, x_desc, hni, marg, bs)
    return jnp.sum(partials)
